# baseline (device time: 129903 ns/iter reference)
import functools

import jax
import jax.numpy as jnp
from jax import lax
from jax.experimental import pallas as pl
from jax.experimental.pallas import tpu as pltpu

N_DEV = 4
N_SUB = 1536


def kernel(A, B):
    m, k_shard = A.shape
    _, n = B.shape
    m_chunk = m // N_DEV
    half = n // 2

    def body(a_ref, b_ref, out_ref, acc, comm_cw, comm_ccw,
             send_cw, recv_cw, send_ccw, recv_ccw):
        my = lax.axis_index("i")
        left = (my + N_DEV - 1) % N_DEV
        right = (my + 1) % N_DEV

        barrier_sem = pltpu.get_barrier_semaphore()
        for nbr in (left, right):
            pl.semaphore_signal(
                barrier_sem, inc=1,
                device_id=(nbr,), device_id_type=pl.DeviceIdType.MESH,
            )
        pl.semaphore_wait(barrier_sem, 2)

        def partial_cols(c, col0, col1, dst_ref, as_bf16):
            a = a_ref[pl.ds(c * m_chunk, m_chunk), :]
            for j in range(col0, col1, N_SUB):
                js = pl.ds(j, N_SUB)
                v = jnp.dot(a, b_ref[:, js], preferred_element_type=jnp.float32)
                dst_ref[:, js] = v.astype(jnp.bfloat16) if as_bf16 else v

        partial_cols((my + N_DEV - 1) % N_DEV, 0, half, out_ref, True)
        partial_cols((my + 1) % N_DEV, half, n, out_ref, True)

        for t in range(N_DEV - 1):
            src_cw = (
                out_ref.at[:, pl.ds(0, half)] if t == 0 else comm_cw.at[t - 1]
            )
            src_ccw = (
                out_ref.at[:, pl.ds(half, half)] if t == 0 else comm_ccw.at[t - 1]
            )
            rdma_cw = pltpu.make_async_remote_copy(
                src_ref=src_cw, dst_ref=comm_cw.at[t],
                send_sem=send_cw.at[t], recv_sem=recv_cw.at[t],
                device_id=(right,), device_id_type=pl.DeviceIdType.MESH,
            )
            rdma_ccw = pltpu.make_async_remote_copy(
                src_ref=src_ccw, dst_ref=comm_ccw.at[t],
                send_sem=send_ccw.at[t], recv_sem=recv_ccw.at[t],
                device_id=(left,), device_id_type=pl.DeviceIdType.MESH,
            )
            rdma_cw.start()
            rdma_ccw.start()

            if t == 1:
                partial_cols((my + 1) % N_DEV, 0, half, acc, False)
                partial_cols((my + N_DEV - 1) % N_DEV, half, n, acc, False)
            else:
                partial_cols((my + 2 - t) % N_DEV, 0, n, acc, False)

            rdma_cw.wait()
            rdma_ccw.wait()

            last = t == N_DEV - 2
            for j in range(0, half, N_SUB):
                js = pl.ds(j, N_SUB)
                js_r = pl.ds(half + j, N_SUB)
                s_cw = comm_cw[t, :, js].astype(jnp.float32) + acc[:, js]
                s_ccw = comm_ccw[t, :, js].astype(jnp.float32) + acc[:, js_r]
                if last:
                    out_ref[:, js] = s_cw.astype(jnp.bfloat16)
                    out_ref[:, js_r] = s_ccw.astype(jnp.bfloat16)
                else:
                    comm_cw[t, :, js] = s_cw.astype(jnp.bfloat16)
                    comm_ccw[t, :, js] = s_ccw.astype(jnp.bfloat16)

        @functools.partial(pl.run_scoped, sem=pltpu.SemaphoreType.REGULAR)
        def _(sem):
            for nbr in (left, right):
                pl.semaphore_signal(
                    sem, inc=1,
                    device_id=(nbr,), device_id_type=pl.DeviceIdType.MESH,
                )
            pl.semaphore_wait(sem, 2)

    call = pl.pallas_call(
        body,
        out_shape=jax.ShapeDtypeStruct((m_chunk, n), jnp.bfloat16),
        in_specs=[
            pl.BlockSpec(memory_space=pltpu.MemorySpace.VMEM),
            pl.BlockSpec(memory_space=pltpu.MemorySpace.VMEM),
        ],
        out_specs=pl.BlockSpec(memory_space=pltpu.MemorySpace.VMEM),
        scratch_shapes=[
            pltpu.VMEM((m_chunk, n), jnp.float32),
            pltpu.VMEM((N_DEV - 1, m_chunk, half), jnp.bfloat16),
            pltpu.VMEM((N_DEV - 1, m_chunk, half), jnp.bfloat16),
            pltpu.SemaphoreType.DMA((N_DEV - 1,)),
            pltpu.SemaphoreType.DMA((N_DEV - 1,)),
            pltpu.SemaphoreType.DMA((N_DEV - 1,)),
            pltpu.SemaphoreType.DMA((N_DEV - 1,)),
        ],
        compiler_params=pltpu.CompilerParams(
            collective_id=0,
            vmem_limit_bytes=56 * 1024 * 1024,
        ),
    )
    return call(A.astype(jnp.bfloat16), B.astype(jnp.bfloat16))


# device time: 120016 ns/iter; 1.0824x vs baseline; 1.0824x over previous
import functools

import jax
import jax.numpy as jnp
from jax import lax
from jax.experimental import pallas as pl
from jax.experimental.pallas import tpu as pltpu

N_DEV = 4
N_RING = 4
RING_ORDER = (0, 2, 1, 3)


def kernel(A, B):
    m, k_shard = A.shape
    _, n = B.shape
    m_chunk = m // N_DEV
    blk = n // N_RING

    def body(a_ref, b_ref, out_ref, acc, comm, send_sems, recv_sems):
        my = lax.axis_index("i")
        left = (my + N_DEV - 1) % N_DEV
        right = (my + 1) % N_DEV

        barrier_sem = pltpu.get_barrier_semaphore()
        for nbr in (left, right):
            pl.semaphore_signal(
                barrier_sem, inc=1,
                device_id=(nbr,), device_id_type=pl.DeviceIdType.MESH,
            )
        pl.semaphore_wait(barrier_sem, 2)

        def cols(r):
            return pl.ds(r * blk, blk)

        def dot_block(c, r):
            a = a_ref[pl.ds(c * m_chunk, m_chunk), :]
            return jnp.dot(a, b_ref[:, cols(r)],
                           preferred_element_type=jnp.float32)

        def hop_chunk(r, t):
            if r < 2:
                return (my + N_DEV + 2 - t) % N_DEV
            return (my + 2 + t) % N_DEV

        def rdma(r, t):
            src = out_ref.at[:, cols(r)] if t == 0 else comm.at[r, t - 1]
            return pltpu.make_async_remote_copy(
                src_ref=src, dst_ref=comm.at[r, t],
                send_sem=send_sems.at[r, t], recv_sem=recv_sems.at[r, t],
                device_id=(right if r < 2 else left,),
                device_id_type=pl.DeviceIdType.MESH,
            )

        for r in RING_ORDER:
            out_ref[:, cols(r)] = dot_block(hop_chunk(r, -1), r).astype(
                jnp.bfloat16)
            rdma(r, 0).start()

        for r in RING_ORDER:
            acc[0, :, cols(r)] = dot_block(hop_chunk(r, 0), r).astype(
                jnp.bfloat16)

        for t in range(N_DEV - 1):
            for r in RING_ORDER:
                rdma(r, t).wait()
                s = (comm[r, t].astype(jnp.float32)
                     + acc[t % 2, :, cols(r)].astype(jnp.float32))
                if t == N_DEV - 2:
                    out_ref[:, cols(r)] = s.astype(jnp.bfloat16)
                else:
                    comm[r, t] = s.astype(jnp.bfloat16)
                    rdma(r, t + 1).start()
            if t < N_DEV - 2:
                for r in RING_ORDER:
                    acc[(t + 1) % 2, :, cols(r)] = dot_block(
                        hop_chunk(r, t + 1), r).astype(jnp.bfloat16)

        @functools.partial(pl.run_scoped, sem=pltpu.SemaphoreType.REGULAR)
        def _(sem):
            for nbr in (left, right):
                pl.semaphore_signal(
                    sem, inc=1,
                    device_id=(nbr,), device_id_type=pl.DeviceIdType.MESH,
                )
            pl.semaphore_wait(sem, 2)

    call = pl.pallas_call(
        body,
        out_shape=jax.ShapeDtypeStruct((m_chunk, n), jnp.bfloat16),
        in_specs=[
            pl.BlockSpec(memory_space=pltpu.MemorySpace.VMEM),
            pl.BlockSpec(memory_space=pltpu.MemorySpace.VMEM),
        ],
        out_specs=pl.BlockSpec(memory_space=pltpu.MemorySpace.VMEM),
        scratch_shapes=[
            pltpu.VMEM((2, m_chunk, n), jnp.bfloat16),
            pltpu.VMEM((N_RING, N_DEV - 1, m_chunk, blk), jnp.bfloat16),
            pltpu.SemaphoreType.DMA((N_RING, N_DEV - 1)),
            pltpu.SemaphoreType.DMA((N_RING, N_DEV - 1)),
        ],
        compiler_params=pltpu.CompilerParams(
            collective_id=0,
            vmem_limit_bytes=56 * 1024 * 1024,
        ),
    )
    return call(A.astype(jnp.bfloat16), B.astype(jnp.bfloat16))


# device time: 109145 ns/iter; 1.1902x vs baseline; 1.0996x over previous
import functools

import jax
import jax.numpy as jnp
from jax import lax
from jax.experimental import pallas as pl
from jax.experimental.pallas import tpu as pltpu

N_DEV = 4
N_RING = 4
RING_ORDER = (0, 2, 1, 3)
A_SLOT = {3: 0, 1: 1, 2: 2, 0: 3}


def kernel(A, B):
    m, k_shard = A.shape
    _, n = B.shape
    m_chunk = m // N_DEV
    blk = n // N_RING

    def body(a_hbm, b_hbm, out_ref,
             a_bf, b_stage, b_bf, acc, comm,
             a_sem, b_sem, send_sems, recv_sems):
        my = lax.axis_index("i")
        left = (my + N_DEV - 1) % N_DEV
        right = (my + 1) % N_DEV

        barrier_sem = pltpu.get_barrier_semaphore()
        for nbr in (left, right):
            pl.semaphore_signal(
                barrier_sem, inc=1,
                device_id=(nbr,), device_id_type=pl.DeviceIdType.MESH,
            )
        pl.semaphore_wait(barrier_sem, 2)

        def cols(r):
            return pl.ds(r * blk, blk)

        def a_dma(off):
            slot = A_SLOT[off]
            return pltpu.make_async_copy(
                a_hbm.at[pl.ds(((my + off) % N_DEV) * m_chunk, m_chunk), :],
                a_bf.at[slot], a_sem.at[slot])

        B_SLOT = {r: i % 2 for i, r in enumerate(RING_ORDER)}

        def b_dma(r):
            return pltpu.make_async_copy(
                b_hbm.at[:, cols(r)], b_stage.at[B_SLOT[r]],
                b_sem.at[B_SLOT[r]])

        a_dma(3).start()
        a_dma(1).start()
        a_dma(2).start()
        b_dma(RING_ORDER[0]).start()
        b_dma(RING_ORDER[1]).start()

        def dot_block(off, r):
            return jnp.dot(a_bf[A_SLOT[off]], b_bf[:, cols(r)],
                           preferred_element_type=jnp.float32)

        def hop_off(r, t):
            return (2 - t) % N_DEV if r < 2 else (2 + t) % N_DEV

        def rdma(r, t):
            src = out_ref.at[:, cols(r)] if t == 0 else comm.at[r, t - 1]
            return pltpu.make_async_remote_copy(
                src_ref=src, dst_ref=comm.at[r, t],
                send_sem=send_sems.at[r, t], recv_sem=recv_sems.at[r, t],
                device_id=(right if r < 2 else left,),
                device_id_type=pl.DeviceIdType.MESH,
            )

        a_dma(3).wait()
        a_dma(1).wait()

        for i, r in enumerate(RING_ORDER):
            b_dma(r).wait()
            b_bf[:, cols(r)] = b_stage[B_SLOT[r]].astype(jnp.bfloat16)
            if i < 2:
                b_dma(RING_ORDER[i + 2]).start()
            out_ref[:, cols(r)] = dot_block(hop_off(r, -1), r).astype(
                jnp.bfloat16)
            rdma(r, 0).start()

        a_dma(2).wait()
        a_dma(0).start()
        for r in RING_ORDER:
            acc[0, :, cols(r)] = dot_block(hop_off(r, 0), r).astype(
                jnp.bfloat16)
        a_dma(0).wait()

        for t in range(N_DEV - 1):
            for r in RING_ORDER:
                rdma(r, t).wait()
                s = (comm[r, t].astype(jnp.float32)
                     + acc[t % 2, :, cols(r)].astype(jnp.float32))
                if t == N_DEV - 2:
                    out_ref[:, cols(r)] = s.astype(jnp.bfloat16)
                else:
                    comm[r, t] = s.astype(jnp.bfloat16)
                    rdma(r, t + 1).start()
            if t < N_DEV - 2:
                for r in RING_ORDER:
                    acc[(t + 1) % 2, :, cols(r)] = dot_block(
                        hop_off(r, t + 1), r).astype(jnp.bfloat16)

        @functools.partial(pl.run_scoped, sem=pltpu.SemaphoreType.REGULAR)
        def _(sem):
            for nbr in (left, right):
                pl.semaphore_signal(
                    sem, inc=1,
                    device_id=(nbr,), device_id_type=pl.DeviceIdType.MESH,
                )
            pl.semaphore_wait(sem, 2)

    call = pl.pallas_call(
        body,
        out_shape=jax.ShapeDtypeStruct((m_chunk, n), jnp.bfloat16),
        in_specs=[
            pl.BlockSpec(memory_space=pl.ANY),
            pl.BlockSpec(memory_space=pl.ANY),
        ],
        out_specs=pl.BlockSpec(memory_space=pltpu.MemorySpace.VMEM),
        scratch_shapes=[
            pltpu.VMEM((N_DEV, m_chunk, k_shard), jnp.bfloat16),
            pltpu.VMEM((2, k_shard, blk), jnp.float32),
            pltpu.VMEM((k_shard, n), jnp.bfloat16),
            pltpu.VMEM((2, m_chunk, n), jnp.bfloat16),
            pltpu.VMEM((N_RING, N_DEV - 1, m_chunk, blk), jnp.bfloat16),
            pltpu.SemaphoreType.DMA((N_DEV,)),
            pltpu.SemaphoreType.DMA((2,)),
            pltpu.SemaphoreType.DMA((N_RING, N_DEV - 1)),
            pltpu.SemaphoreType.DMA((N_RING, N_DEV - 1)),
        ],
        compiler_params=pltpu.CompilerParams(
            collective_id=0,
            vmem_limit_bytes=58 * 1024 * 1024,
        ),
    )
    return call(A.astype(jnp.bfloat16), B)


# device time: 105818 ns/iter; 1.2276x vs baseline; 1.0314x over previous
import functools

import jax
import jax.numpy as jnp
from jax import lax
from jax.experimental import pallas as pl
from jax.experimental.pallas import tpu as pltpu

N_DEV = 4
N_RING = 4
RING_ORDER = (0, 2, 1, 3)
A_SLOT = {3: 0, 1: 1, 2: 2, 0: 3}
A_ORDER = (3, 1, 2, 0)


def kernel(A, B):
    m, k_shard = A.shape
    _, n = B.shape
    m_chunk = m // N_DEV
    blk = n // N_RING

    def body(a_hbm, b_hbm, out_ref,
             a_stage, a_bf, b_stage, b_bf, acc, comm,
             a_sem, b_sem, send_sems, recv_sems):
        my = lax.axis_index("i")
        left = (my + N_DEV - 1) % N_DEV
        right = (my + 1) % N_DEV

        barrier_sem = pltpu.get_barrier_semaphore()
        for nbr in (left, right):
            pl.semaphore_signal(
                barrier_sem, inc=1,
                device_id=(nbr,), device_id_type=pl.DeviceIdType.MESH,
            )
        pl.semaphore_wait(barrier_sem, 2)

        def cols(r):
            return pl.ds(r * blk, blk)

        def a_dma(off):
            return pltpu.make_async_copy(
                a_hbm.at[pl.ds(((my + off) % N_DEV) * m_chunk, m_chunk), :],
                a_stage, a_sem)

        def b_dma(r):
            return pltpu.make_async_copy(b_hbm.at[:, cols(r)], b_stage, b_sem)

        def a_step(i):
            off = A_ORDER[i]
            a_dma(off).wait()
            a_bf[A_SLOT[off]] = a_stage[...].astype(jnp.bfloat16)
            if i + 1 < len(A_ORDER):
                a_dma(A_ORDER[i + 1]).start()

        def b_step(i):
            r = RING_ORDER[i]
            b_dma(r).wait()
            b_bf[:, cols(r)] = b_stage[...].astype(jnp.bfloat16)
            if i + 1 < len(RING_ORDER):
                b_dma(RING_ORDER[i + 1]).start()

        def dot_block(off, r):
            return jnp.dot(a_bf[A_SLOT[off]], b_bf[:, cols(r)],
                           preferred_element_type=jnp.float32)

        def hop_off(r, t):
            return (2 - t) % N_DEV if r < 2 else (2 + t) % N_DEV

        def rdma(r, t):
            src = out_ref.at[:, cols(r)] if t == 0 else comm.at[r, t - 1]
            return pltpu.make_async_remote_copy(
                src_ref=src, dst_ref=comm.at[r, t],
                send_sem=send_sems.at[r, t], recv_sem=recv_sems.at[r, t],
                device_id=(right if r < 2 else left,),
                device_id_type=pl.DeviceIdType.MESH,
            )

        a_dma(A_ORDER[0]).start()
        b_dma(RING_ORDER[0]).start()
        a_step(0)

        for i, r in enumerate(RING_ORDER):
            b_step(i)
            out_ref[:, cols(r)] = dot_block(hop_off(r, -1), r).astype(
                jnp.bfloat16)
            rdma(r, 0).start()
            if i < 3:
                a_step(i + 1)

        for r in RING_ORDER:
            acc[0, :, cols(r)] = dot_block(hop_off(r, 0), r).astype(
                jnp.bfloat16)

        for t in range(N_DEV - 1):
            for r in RING_ORDER:
                rdma(r, t).wait()
                s = comm[r, t] + acc[t % 2, :, cols(r)]
                if t == N_DEV - 2:
                    out_ref[:, cols(r)] = s
                else:
                    comm[r, t] = s
                    rdma(r, t + 1).start()
            if t < N_DEV - 2:
                for r in RING_ORDER:
                    acc[(t + 1) % 2, :, cols(r)] = dot_block(
                        hop_off(r, t + 1), r).astype(jnp.bfloat16)

        @functools.partial(pl.run_scoped, sem=pltpu.SemaphoreType.REGULAR)
        def _(sem):
            for nbr in (left, right):
                pl.semaphore_signal(
                    sem, inc=1,
                    device_id=(nbr,), device_id_type=pl.DeviceIdType.MESH,
                )
            pl.semaphore_wait(sem, 2)

    call = pl.pallas_call(
        body,
        out_shape=jax.ShapeDtypeStruct((m_chunk, n), jnp.bfloat16),
        in_specs=[
            pl.BlockSpec(memory_space=pl.ANY),
            pl.BlockSpec(memory_space=pl.ANY),
        ],
        out_specs=pl.BlockSpec(memory_space=pltpu.MemorySpace.VMEM),
        scratch_shapes=[
            pltpu.VMEM((m_chunk, k_shard), jnp.float32),
            pltpu.VMEM((N_DEV, m_chunk, k_shard), jnp.bfloat16),
            pltpu.VMEM((k_shard, blk), jnp.float32),
            pltpu.VMEM((k_shard, n), jnp.bfloat16),
            pltpu.VMEM((2, m_chunk, n), jnp.bfloat16),
            pltpu.VMEM((N_RING, N_DEV - 1, m_chunk, blk), jnp.bfloat16),
            pltpu.SemaphoreType.DMA,
            pltpu.SemaphoreType.DMA,
            pltpu.SemaphoreType.DMA((N_RING, N_DEV - 1)),
            pltpu.SemaphoreType.DMA((N_RING, N_DEV - 1)),
        ],
        compiler_params=pltpu.CompilerParams(
            collective_id=0,
            vmem_limit_bytes=58 * 1024 * 1024,
        ),
    )
    return call(A, B)


# device time: 101562 ns/iter; 1.2791x vs baseline; 1.0419x over previous
import functools

import jax
import jax.numpy as jnp
from jax import lax
from jax.experimental import pallas as pl
from jax.experimental.pallas import tpu as pltpu

N_DEV = 4
N_RING = 4
RING_ORDER = (0, 2, 1, 3)
A_SLOT = {3: 0, 1: 1, 2: 2, 0: 3}
A_ORDER = (3, 1, 2, 0)


def kernel(A, B):
    m, k_shard = A.shape
    _, n = B.shape
    m_chunk = m // N_DEV
    blk = n // N_RING

    def body(a_hbm, b_hbm, out_ref,
             a_stage, a_bf, b_stage, b_bf, acc, comm,
             a_sem, b_sem, send_sems, recv_sems):
        my = lax.axis_index("i")
        left = (my + N_DEV - 1) % N_DEV
        right = (my + 1) % N_DEV

        barrier_sem = pltpu.get_barrier_semaphore()
        for nbr in (left, right):
            pl.semaphore_signal(
                barrier_sem, inc=1,
                device_id=(nbr,), device_id_type=pl.DeviceIdType.MESH,
            )
        pl.semaphore_wait(barrier_sem, 2)

        def cols(r):
            return pl.ds(r * blk, blk)

        def a_dma(off):
            return pltpu.make_async_copy(
                a_hbm.at[pl.ds(((my + off) % N_DEV) * m_chunk, m_chunk), :],
                a_stage, a_sem)

        def b_dma(r):
            return pltpu.make_async_copy(b_hbm.at[:, cols(r)], b_stage, b_sem)

        def a_step(i):
            off = A_ORDER[i]
            a_dma(off).wait()
            a_bf[A_SLOT[off]] = a_stage[...].astype(jnp.bfloat16)
            if i + 1 < len(A_ORDER):
                a_dma(A_ORDER[i + 1]).start()

        def b_step(i):
            r = RING_ORDER[i]
            b_dma(r).wait()
            b_bf[:, cols(r)] = b_stage[...].astype(jnp.bfloat16)
            if i + 1 < len(RING_ORDER):
                b_dma(RING_ORDER[i + 1]).start()

        def dot_block(off, r):
            return jnp.dot(a_bf[A_SLOT[off]], b_bf[:, cols(r)],
                           preferred_element_type=jnp.float32)

        def hop_off(r, t):
            return (2 - t) % N_DEV if r < 2 else (2 + t) % N_DEV

        def rdma(r, t):
            src = out_ref.at[:, cols(r)] if t == 0 else comm.at[r, t - 1]
            return pltpu.make_async_remote_copy(
                src_ref=src, dst_ref=comm.at[r, t],
                send_sem=send_sems.at[r, t], recv_sem=recv_sems.at[r, t],
                device_id=(right if r < 2 else left,),
                device_id_type=pl.DeviceIdType.MESH,
            )

        a_dma(A_ORDER[0]).start()
        b_dma(RING_ORDER[0]).start()
        a_step(0)

        for i, r in enumerate(RING_ORDER):
            b_step(i)
            out_ref[:, cols(r)] = dot_block(hop_off(r, -1), r).astype(
                jnp.bfloat16)
            rdma(r, 0).start()
            if i < 3:
                a_step(i + 1)

        pairs = ((0, 2), (1, 3))
        for r in pairs[0]:
            acc[0, :, cols(r)] = dot_block(hop_off(r, 0), r).astype(
                jnp.bfloat16)

        for t in range(N_DEV - 1):
            for pi, pair in enumerate(pairs):
                for r in pair:
                    rdma(r, t).wait()
                    s = comm[r, t] + acc[t % 2, :, cols(r)]
                    if t == N_DEV - 2:
                        out_ref[:, cols(r)] = s
                    else:
                        comm[r, t] = s
                        rdma(r, t + 1).start()
                if t == 0 and pi == 0:
                    for r in pairs[1]:
                        acc[0, :, cols(r)] = dot_block(
                            hop_off(r, 0), r).astype(jnp.bfloat16)
                if t < N_DEV - 2:
                    for r in pair:
                        acc[(t + 1) % 2, :, cols(r)] = dot_block(
                            hop_off(r, t + 1), r).astype(jnp.bfloat16)

        @functools.partial(pl.run_scoped, sem=pltpu.SemaphoreType.REGULAR)
        def _(sem):
            for nbr in (left, right):
                pl.semaphore_signal(
                    sem, inc=1,
                    device_id=(nbr,), device_id_type=pl.DeviceIdType.MESH,
                )
            pl.semaphore_wait(sem, 2)

    call = pl.pallas_call(
        body,
        out_shape=jax.ShapeDtypeStruct((m_chunk, n), jnp.bfloat16),
        in_specs=[
            pl.BlockSpec(memory_space=pl.ANY),
            pl.BlockSpec(memory_space=pl.ANY),
        ],
        out_specs=pl.BlockSpec(memory_space=pltpu.MemorySpace.VMEM),
        scratch_shapes=[
            pltpu.VMEM((m_chunk, k_shard), jnp.float32),
            pltpu.VMEM((N_DEV, m_chunk, k_shard), jnp.bfloat16),
            pltpu.VMEM((k_shard, blk), jnp.float32),
            pltpu.VMEM((k_shard, n), jnp.bfloat16),
            pltpu.VMEM((2, m_chunk, n), jnp.bfloat16),
            pltpu.VMEM((N_RING, N_DEV - 1, m_chunk, blk), jnp.bfloat16),
            pltpu.SemaphoreType.DMA,
            pltpu.SemaphoreType.DMA,
            pltpu.SemaphoreType.DMA((N_RING, N_DEV - 1)),
            pltpu.SemaphoreType.DMA((N_RING, N_DEV - 1)),
        ],
        compiler_params=pltpu.CompilerParams(
            collective_id=0,
            vmem_limit_bytes=58 * 1024 * 1024,
        ),
    )
    return call(A, B)
